# baseline (speedup 1.0000x reference)
import functools

import jax
import jax.numpy as jnp
from jax import lax
from jax.experimental import pallas as pl
from jax.experimental.pallas import tpu as pltpu
from jax.experimental.pallas import tpu_sc as plsc

N = 10000
D = 128
E = 320000
NCLS = 64
NC, NS = 2, 16
NW = NC * NS
C = 64
CPW = 160
CPW0, CPW1 = 288, 32
GRP = 32
NGRP = CPW // GRP
NCH = CPW * NW
EP = NCH * C
NP = 10112
RPT = NP // NS
NZB = (RPT + C - 1) // C

_mesh = plsc.VectorSubcoreMesh(core_axis_name="c", subcore_axis_name="s")


@functools.partial(
    pl.kernel,
    out_type=jax.ShapeDtypeStruct((NC, NP, D), jnp.float32),
    mesh=_mesh,
    scratch_types=[
        pltpu.VMEM((GRP, C), jnp.int32),
        pltpu.VMEM((GRP, C), jnp.int32),
        pltpu.VMEM((C, D), jnp.float32),
        pltpu.VMEM((C, D), jnp.float32),
        pltpu.VMEM_SHARED((NP, D), jnp.float32),
        pltpu.SemaphoreType.DMA,
        pltpu.SemaphoreType.DMA,
        pltpu.SemaphoreType.DMA,
        pltpu.SemaphoreType.DMA,
    ],
)
def _sc_aggregate(x_hbm, srcs_hbm, dsts_hbm, zeros_hbm, agg_out,
                  src_v, dst_v, rows0, rows1, agg_sh,
                  gsem0, gsem1, ssem0, ssem1):
    cid = lax.axis_index("c")
    sid = lax.axis_index("s")
    r0 = sid * RPT
    pltpu.sync_copy(zeros_hbm, rows0)
    for k in range(NZB):
        sz = min(C, RPT - k * C)
        pltpu.sync_copy(rows0.at[pl.ds(0, sz)], agg_sh.at[pl.ds(r0 + k * C, sz)])
    c0 = jnp.where(cid == 0, sid * CPW0, NS * CPW0 + sid * CPW1)
    ng = jnp.where(cid == 0, CPW0 // GRP, CPW1 // GRP)
    plsc.subcore_barrier()

    def group(g, carry):
        pltpu.sync_copy(srcs_hbm.at[pl.ds(c0 + g * GRP, GRP)], src_v)
        pltpu.sync_copy(dsts_hbm.at[pl.ds(c0 + g * GRP, GRP)], dst_v)
        pltpu.async_copy(x_hbm.at[src_v.at[0]], rows0, gsem0)

        def pair(p, c2):
            pltpu.make_async_copy(x_hbm.at[src_v.at[0]], rows0, gsem0).wait()
            pltpu.async_copy(x_hbm.at[src_v.at[2 * p + 1]], rows1, gsem1)
            pltpu.async_copy(rows0, agg_sh.at[dst_v.at[2 * p]], ssem0, add=True)
            pltpu.make_async_copy(x_hbm.at[src_v.at[0]], rows1, gsem1).wait()
            pltpu.make_async_copy(rows0, agg_sh.at[dst_v.at[0]], ssem0).wait()
            nxt = jnp.minimum(2 * p + 2, GRP - 1)
            pltpu.async_copy(x_hbm.at[src_v.at[nxt]], rows0, gsem0)
            pltpu.async_copy(rows1, agg_sh.at[dst_v.at[2 * p + 1]], ssem1, add=True)
            pltpu.make_async_copy(rows1, agg_sh.at[dst_v.at[0]], ssem1).wait()
            return c2

        lax.fori_loop(0, GRP // 2, pair, 0)
        pltpu.make_async_copy(x_hbm.at[src_v.at[0]], rows0, gsem0).wait()
        return carry

    lax.fori_loop(0, ng, group, 0)
    plsc.subcore_barrier()
    for k in range(NZB):
        sz = min(C, RPT - k * C)
        pltpu.sync_copy(agg_sh.at[pl.ds(r0 + k * C, sz)], rows0.at[pl.ds(0, sz)])
        pltpu.sync_copy(rows0.at[pl.ds(0, sz)], agg_out.at[cid, pl.ds(r0 + k * C, sz)])


@functools.partial(
    pl.kernel,
    out_type=jax.ShapeDtypeStruct((NC, NP, D), jnp.float32),
    mesh=_mesh,
    scratch_types=[
        pltpu.VMEM((GRP, C), jnp.int32),
        pltpu.VMEM((C, D), jnp.float32),
        pltpu.VMEM_SHARED((NP, D), jnp.float32),
        pltpu.SemaphoreType.DMA,
    ],
)
def _sc_count(dsts_hbm, zeros_hbm, ones_hbm, cnt_out, dst_v, ones_v, cnt_sh, sem):
    cid = lax.axis_index("c")
    sid = lax.axis_index("s")
    wid = cid * NS + sid
    r0 = sid * RPT
    pltpu.sync_copy(zeros_hbm, ones_v)
    for k in range(NZB):
        sz = min(C, RPT - k * C)
        pltpu.sync_copy(ones_v.at[pl.ds(0, sz)], cnt_sh.at[pl.ds(r0 + k * C, sz)])
    pltpu.sync_copy(ones_hbm, ones_v)
    c0 = wid * CPW
    plsc.subcore_barrier()

    def group(g, carry):
        pltpu.sync_copy(dsts_hbm.at[pl.ds(c0 + g * GRP, GRP)], dst_v)

        def fire(j, c2):
            pltpu.async_copy(ones_v, cnt_sh.at[dst_v.at[j]], sem, add=True)
            return c2

        def drain(j, c2):
            pltpu.make_async_copy(ones_v, cnt_sh.at[dst_v.at[0]], sem).wait()
            return c2

        lax.fori_loop(0, GRP, fire, 0)
        lax.fori_loop(0, GRP, drain, 0)
        return carry

    lax.fori_loop(0, NGRP, group, 0)
    plsc.subcore_barrier()
    for k in range(NZB):
        sz = min(C, RPT - k * C)
        pltpu.sync_copy(cnt_sh.at[pl.ds(r0 + k * C, sz)], ones_v.at[pl.ds(0, sz)])
        pltpu.sync_copy(ones_v.at[pl.ds(0, sz)], cnt_out.at[cid, pl.ds(r0 + k * C, sz)])


def _tc_dense(agg, cnt, xin, w_l, b_l, w_r, w_c=None, b_c=None):
    classifier = w_c is not None
    br = 2000
    out_d = NCLS if classifier else D

    def body(agg_ref, cnt_ref, x_ref, wl_ref, bl_ref, wr_ref, *rest):
        out_ref = rest[-1]
        a = agg_ref[0] + agg_ref[1]
        c = cnt_ref[0, :, :1] + cnt_ref[1, :, :1]
        mean = a / jnp.maximum(c, 1.0)
        h = mean @ wl_ref[...] + bl_ref[...] + x_ref[...] @ wr_ref[...]
        h = jnp.maximum(h, 0.0)
        if classifier:
            out_ref[...] = h @ rest[0][...] + rest[1][...]
        else:
            out_ref[...] = h

    in_specs = [
        pl.BlockSpec((NC, br, D), lambda i: (0, i, 0)),
        pl.BlockSpec((NC, br, D), lambda i: (0, i, 0)),
        pl.BlockSpec((br, D), lambda i: (i, 0)),
        pl.BlockSpec((D, D), lambda i: (0, 0)),
        pl.BlockSpec((1, D), lambda i: (0, 0)),
        pl.BlockSpec((D, D), lambda i: (0, 0)),
    ]
    args = [agg, cnt, xin, w_l, b_l.reshape(1, D), w_r]
    if classifier:
        in_specs += [pl.BlockSpec((D, NCLS), lambda i: (0, 0)),
                     pl.BlockSpec((1, NCLS), lambda i: (0, 0))]
        args += [w_c, b_c.reshape(1, NCLS)]
    return pl.pallas_call(
        body,
        grid=(N // br,),
        in_specs=in_specs,
        out_specs=pl.BlockSpec((br, out_d), lambda i: (i, 0)),
        out_shape=jax.ShapeDtypeStruct((N, out_d), jnp.float32),
    )(*args)


def kernel(x, edge_index, edge_weight, W_l0, b_l0, W_r0, W_l1, b_l1, W_r1, Wc, bc):
    src = edge_index[0].astype(jnp.int32)
    dst = edge_index[1].astype(jnp.int32)
    pad = EP - E
    pad_dst = N + (jnp.arange(pad, dtype=jnp.int32) % (NP - N))
    srcs = jnp.concatenate([src, jnp.zeros((pad,), jnp.int32)]).reshape(NCH, C)
    dsts = jnp.concatenate([dst, pad_dst]).reshape(NCH, C)
    zeros = jnp.zeros((C, D), jnp.float32)
    ones = jnp.ones((C, D), jnp.float32)

    cnt0 = _sc_count(dsts, zeros, ones)
    agg0 = _sc_aggregate(x, srcs, dsts, zeros)
    h = _tc_dense(agg0, cnt0, x, W_l0, b_l0, W_r0)
    agg1 = _sc_aggregate(h, srcs, dsts, zeros)
    out = _tc_dense(agg1, cnt0, h, W_l1, b_l1, W_r1, Wc, bc)
    return out


# speedup vs baseline: 1.0259x; 1.0259x over previous
import functools

import jax
import jax.numpy as jnp
from jax import lax
from jax.experimental import pallas as pl
from jax.experimental.pallas import tpu as pltpu
from jax.experimental.pallas import tpu_sc as plsc

N = 10000
D = 128
E = 320000
NCLS = 64
NC, NS = 2, 16
NW = NC * NS
C = 64
CPW = 160
CPW0, CPW1 = 256, 64
GRP = 32
NGRP = CPW // GRP
NCH = CPW * NW
EP = NCH * C
NP = 10112
RPT = NP // NS
NZB = (RPT + C - 1) // C

_mesh = plsc.VectorSubcoreMesh(core_axis_name="c", subcore_axis_name="s")


@functools.partial(
    pl.kernel,
    out_type=jax.ShapeDtypeStruct((NC, NP, D), jnp.float32),
    mesh=_mesh,
    scratch_types=[
        pltpu.VMEM((GRP, C), jnp.int32),
        pltpu.VMEM((GRP, C), jnp.int32),
        pltpu.VMEM((C, D), jnp.float32),
        pltpu.VMEM((C, D), jnp.float32),
        pltpu.VMEM_SHARED((NP, D), jnp.float32),
        pltpu.SemaphoreType.DMA,
        pltpu.SemaphoreType.DMA,
        pltpu.SemaphoreType.DMA,
        pltpu.SemaphoreType.DMA,
    ],
)
def _sc_aggregate(x_hbm, srcs_hbm, dsts_hbm, zeros_hbm, agg_out,
                  src_v, dst_v, rows0, rows1, agg_sh,
                  gsem0, gsem1, ssem0, ssem1):
    cid = lax.axis_index("c")
    sid = lax.axis_index("s")
    r0 = sid * RPT
    pltpu.sync_copy(zeros_hbm, rows0)
    for k in range(NZB):
        sz = min(C, RPT - k * C)
        pltpu.sync_copy(rows0.at[pl.ds(0, sz)], agg_sh.at[pl.ds(r0 + k * C, sz)])
    c0 = jnp.where(cid == 0, sid * CPW0, NS * CPW0 + sid * CPW1)
    ng = jnp.where(cid == 0, CPW0 // GRP, CPW1 // GRP)
    plsc.subcore_barrier()

    def group(g, carry):
        pltpu.sync_copy(srcs_hbm.at[pl.ds(c0 + g * GRP, GRP)], src_v)
        pltpu.sync_copy(dsts_hbm.at[pl.ds(c0 + g * GRP, GRP)], dst_v)
        pltpu.async_copy(x_hbm.at[src_v.at[0]], rows0, gsem0)

        def pair(p, c2):
            pltpu.make_async_copy(x_hbm.at[src_v.at[0]], rows0, gsem0).wait()
            pltpu.async_copy(x_hbm.at[src_v.at[2 * p + 1]], rows1, gsem1)
            pltpu.async_copy(rows0, agg_sh.at[dst_v.at[2 * p]], ssem0, add=True)
            pltpu.make_async_copy(x_hbm.at[src_v.at[0]], rows1, gsem1).wait()
            pltpu.make_async_copy(rows0, agg_sh.at[dst_v.at[0]], ssem0).wait()
            nxt = jnp.minimum(2 * p + 2, GRP - 1)
            pltpu.async_copy(x_hbm.at[src_v.at[nxt]], rows0, gsem0)
            pltpu.async_copy(rows1, agg_sh.at[dst_v.at[2 * p + 1]], ssem1, add=True)
            pltpu.make_async_copy(rows1, agg_sh.at[dst_v.at[0]], ssem1).wait()
            return c2

        lax.fori_loop(0, GRP // 2, pair, 0)
        pltpu.make_async_copy(x_hbm.at[src_v.at[0]], rows0, gsem0).wait()
        return carry

    lax.fori_loop(0, ng, group, 0)
    plsc.subcore_barrier()
    for k in range(NZB):
        sz = min(C, RPT - k * C)
        pltpu.sync_copy(agg_sh.at[pl.ds(r0 + k * C, sz)], rows0.at[pl.ds(0, sz)])
        pltpu.sync_copy(rows0.at[pl.ds(0, sz)], agg_out.at[cid, pl.ds(r0 + k * C, sz)])


@functools.partial(
    pl.kernel,
    out_type=jax.ShapeDtypeStruct((NC, NP, D), jnp.float32),
    mesh=_mesh,
    scratch_types=[
        pltpu.VMEM((GRP, C), jnp.int32),
        pltpu.VMEM((C, D), jnp.float32),
        pltpu.VMEM_SHARED((NP, D), jnp.float32),
        pltpu.SemaphoreType.DMA,
    ],
)
def _sc_count(dsts_hbm, zeros_hbm, ones_hbm, cnt_out, dst_v, ones_v, cnt_sh, sem):
    cid = lax.axis_index("c")
    sid = lax.axis_index("s")
    wid = cid * NS + sid
    r0 = sid * RPT
    pltpu.sync_copy(zeros_hbm, ones_v)
    for k in range(NZB):
        sz = min(C, RPT - k * C)
        pltpu.sync_copy(ones_v.at[pl.ds(0, sz)], cnt_sh.at[pl.ds(r0 + k * C, sz)])
    pltpu.sync_copy(ones_hbm, ones_v)
    c0 = wid * CPW
    plsc.subcore_barrier()

    def group(g, carry):
        pltpu.sync_copy(dsts_hbm.at[pl.ds(c0 + g * GRP, GRP)], dst_v)

        def fire(j, c2):
            pltpu.async_copy(ones_v, cnt_sh.at[dst_v.at[j]], sem, add=True)
            return c2

        def drain(j, c2):
            pltpu.make_async_copy(ones_v, cnt_sh.at[dst_v.at[0]], sem).wait()
            return c2

        lax.fori_loop(0, GRP, fire, 0)
        lax.fori_loop(0, GRP, drain, 0)
        return carry

    lax.fori_loop(0, NGRP, group, 0)
    plsc.subcore_barrier()
    for k in range(NZB):
        sz = min(C, RPT - k * C)
        pltpu.sync_copy(cnt_sh.at[pl.ds(r0 + k * C, sz)], ones_v.at[pl.ds(0, sz)])
        pltpu.sync_copy(ones_v.at[pl.ds(0, sz)], cnt_out.at[cid, pl.ds(r0 + k * C, sz)])


def _tc_dense(agg, cnt, xin, w_l, b_l, w_r, w_c=None, b_c=None):
    classifier = w_c is not None
    br = 2000
    out_d = NCLS if classifier else D

    def body(agg_ref, cnt_ref, x_ref, wl_ref, bl_ref, wr_ref, *rest):
        out_ref = rest[-1]
        a = agg_ref[0] + agg_ref[1]
        c = cnt_ref[0, :, :1] + cnt_ref[1, :, :1]
        mean = a / jnp.maximum(c, 1.0)
        h = mean @ wl_ref[...] + bl_ref[...] + x_ref[...] @ wr_ref[...]
        h = jnp.maximum(h, 0.0)
        if classifier:
            out_ref[...] = h @ rest[0][...] + rest[1][...]
        else:
            out_ref[...] = h

    in_specs = [
        pl.BlockSpec((NC, br, D), lambda i: (0, i, 0)),
        pl.BlockSpec((NC, br, D), lambda i: (0, i, 0)),
        pl.BlockSpec((br, D), lambda i: (i, 0)),
        pl.BlockSpec((D, D), lambda i: (0, 0)),
        pl.BlockSpec((1, D), lambda i: (0, 0)),
        pl.BlockSpec((D, D), lambda i: (0, 0)),
    ]
    args = [agg, cnt, xin, w_l, b_l.reshape(1, D), w_r]
    if classifier:
        in_specs += [pl.BlockSpec((D, NCLS), lambda i: (0, 0)),
                     pl.BlockSpec((1, NCLS), lambda i: (0, 0))]
        args += [w_c, b_c.reshape(1, NCLS)]
    return pl.pallas_call(
        body,
        grid=(N // br,),
        in_specs=in_specs,
        out_specs=pl.BlockSpec((br, out_d), lambda i: (i, 0)),
        out_shape=jax.ShapeDtypeStruct((N, out_d), jnp.float32),
    )(*args)


def kernel(x, edge_index, edge_weight, W_l0, b_l0, W_r0, W_l1, b_l1, W_r1, Wc, bc):
    src = edge_index[0].astype(jnp.int32)
    dst = edge_index[1].astype(jnp.int32)
    pad = EP - E
    pad_dst = N + (jnp.arange(pad, dtype=jnp.int32) % (NP - N))
    srcs = jnp.concatenate([src, jnp.zeros((pad,), jnp.int32)]).reshape(NCH, C)
    dsts = jnp.concatenate([dst, pad_dst]).reshape(NCH, C)
    zeros = jnp.zeros((C, D), jnp.float32)
    ones = jnp.ones((C, D), jnp.float32)

    cnt0 = _sc_count(dsts, zeros, ones)
    agg0 = _sc_aggregate(x, srcs, dsts, zeros)
    h = _tc_dense(agg0, cnt0, x, W_l0, b_l0, W_r0)
    agg1 = _sc_aggregate(h, srcs, dsts, zeros)
    out = _tc_dense(agg1, cnt0, h, W_l1, b_l1, W_r1, Wc, bc)
    return out
